# Initial kernel scaffold; baseline (speedup 1.0000x reference)
import jax
import jax.numpy as jnp
from jax.experimental import pallas as pl


def kernel(x, embedding_matrix):
    raise NotImplementedError("write your pallas kernel here")


# speedup vs baseline: 1.4759x; 1.4759x over previous
import functools

import jax
import jax.numpy as jnp
from jax import lax
from jax.experimental import pallas as pl
from jax.experimental.pallas import tpu as pltpu
from jax.experimental.pallas import tpu_sc as plsc


@functools.lru_cache(maxsize=None)
def _make_gather(vocab, dim, num_rows):
    info = plsc.get_sparse_core_info()
    num_workers = info.num_cores * info.num_subcores
    assert num_rows % (8 * num_workers) == 0
    rows_per_worker = num_rows // num_workers

    chunk = 1600
    while rows_per_worker % chunk:
        chunk //= 2
    n_chunks = rows_per_worker // chunk

    mesh = plsc.VectorSubcoreMesh(core_axis_name="c", subcore_axis_name="s")

    @functools.partial(
        pl.kernel,
        out_type=jax.ShapeDtypeStruct((num_rows, dim), jnp.float32),
        mesh=mesh,
        compiler_params=pltpu.CompilerParams(use_tc_tiling_on_sc=False),
        scratch_types=[
            pltpu.VMEM((chunk,), jnp.int32),
            pltpu.VMEM((chunk, dim), jnp.float32),
            pltpu.SemaphoreType.DMA,
        ],
    )
    def gather_kernel(table_hbm, idx_hbm, out_hbm, idx_v, rows_v, sem):
        wid = lax.axis_index("s") * info.num_cores + lax.axis_index("c")
        base = wid * rows_per_worker

        def body(i, carry):
            off = base + i * chunk
            pltpu.sync_copy(idx_hbm.at[pl.ds(off, chunk)], idx_v)
            pltpu.async_copy(table_hbm.at[idx_v], rows_v, sem).wait()
            pltpu.sync_copy(rows_v, out_hbm.at[pl.ds(off, chunk)])
            return carry

        lax.fori_loop(0, n_chunks, body, 0)

    return gather_kernel


def kernel(x, embedding_matrix):
    batch, hist = x.shape
    vocab, dim = embedding_matrix.shape
    flat_idx = x.reshape(-1)
    gather = _make_gather(vocab, dim, batch * hist)
    out = gather(embedding_matrix, flat_idx)
    return out.reshape(batch, hist, dim)


# speedup vs baseline: 1.4925x; 1.0113x over previous
import functools

import jax
import jax.numpy as jnp
from jax import lax
from jax.experimental import pallas as pl
from jax.experimental.pallas import tpu as pltpu
from jax.experimental.pallas import tpu_sc as plsc


@functools.lru_cache(maxsize=None)
def _make_gather(vocab, dim, num_rows):
    info = plsc.get_sparse_core_info()
    num_workers = info.num_cores * info.num_subcores
    assert num_rows % (8 * num_workers) == 0
    rows_per_worker = num_rows // num_workers

    chunk = 1600
    while rows_per_worker % chunk:
        chunk //= 2
    n_chunks = rows_per_worker // chunk

    mesh = plsc.VectorSubcoreMesh(core_axis_name="c", subcore_axis_name="s")

    @functools.partial(
        pl.kernel,
        out_type=jax.ShapeDtypeStruct((num_rows, dim), jnp.float32),
        mesh=mesh,
        compiler_params=pltpu.CompilerParams(use_tc_tiling_on_sc=False),
        scratch_types=[
            pltpu.VMEM((rows_per_worker,), jnp.int32),
            pltpu.VMEM((chunk, dim), jnp.float32),
            pltpu.VMEM((chunk, dim), jnp.float32),
            pltpu.SemaphoreType.DMA,
            pltpu.SemaphoreType.DMA,
            pltpu.SemaphoreType.DMA,
            pltpu.SemaphoreType.DMA,
        ],
    )
    def gather_kernel(table_hbm, idx_hbm, out_hbm, idx_v, rows0, rows1,
                      sg0, sg1, sw0, sw1):
        rows = [rows0, rows1]
        sg = [sg0, sg1]
        sw = [sw0, sw1]
        wid = lax.axis_index("s") * info.num_cores + lax.axis_index("c")
        base = wid * rows_per_worker

        pltpu.sync_copy(idx_hbm.at[pl.ds(base, rows_per_worker)], idx_v)

        def start_gather(i):
            b = i % 2
            pltpu.async_copy(
                table_hbm.at[idx_v.at[pl.ds(i * chunk, chunk)]], rows[b], sg[b])

        def wait_gather(i):
            b = i % 2
            pltpu.make_async_copy(
                table_hbm.at[idx_v.at[pl.ds(i * chunk, chunk)]], rows[b],
                sg[b]).wait()

        def start_wb(i):
            b = i % 2
            pltpu.async_copy(rows[b], out_hbm.at[pl.ds(base + i * chunk, chunk)],
                             sw[b])

        def wait_wb(i):
            b = i % 2
            pltpu.make_async_copy(
                rows[b], out_hbm.at[pl.ds(base + i * chunk, chunk)],
                sw[b]).wait()

        start_gather(0)
        for i in range(1, n_chunks):
            wait_gather(i - 1)
            start_wb(i - 1)
            if i >= 2:
                wait_wb(i)
            start_gather(i)
        wait_gather(n_chunks - 1)
        start_wb(n_chunks - 1)
        wait_wb(n_chunks - 2)
        wait_wb(n_chunks - 1)

    return gather_kernel


def kernel(x, embedding_matrix):
    batch, hist = x.shape
    vocab, dim = embedding_matrix.shape
    flat_idx = x.reshape(-1)
    gather = _make_gather(vocab, dim, batch * hist)
    out = gather(embedding_matrix, flat_idx)
    return out.reshape(batch, hist, dim)
